# Initial kernel scaffold; baseline (speedup 1.0000x reference)
import jax
import jax.numpy as jnp
from jax.experimental import pallas as pl


def kernel(x, edge_index, W_neigh, b_neigh):
    raise NotImplementedError("write your pallas kernel here")


# speedup vs baseline: 4.0039x; 4.0039x over previous
import functools

import jax
import jax.numpy as jnp
from jax import lax
from jax.experimental import pallas as pl
from jax.experimental.pallas import tpu as pltpu
from jax.experimental.pallas import tpu_sc as plsc

N_NODES = 10000
N_EDGES = 320000
D = 128

NC = 2
NS = 16
CHUNK = 128
BLK = 16
NBLK = 5
NPAD = 10240
ROWS_PER_TILE = NPAD // NS
E_PAD = NC * NS * NBLK * BLK * CHUNK


def _sc_aggregate(x, src_p, dst_p):
  mesh = plsc.VectorSubcoreMesh(core_axis_name="c", subcore_axis_name="s")

  @functools.partial(
      pl.kernel,
      out_type=[
          jax.ShapeDtypeStruct((NC, NPAD, D), jnp.float32),
          jax.ShapeDtypeStruct((NC, NPAD), jnp.float32),
      ],
      mesh=mesh,
      scratch_types=[
          pltpu.VMEM_SHARED((NPAD, D), jnp.float32),
          pltpu.VMEM_SHARED((NPAD,), jnp.float32),
          pltpu.VMEM((2, BLK, CHUNK), jnp.int32),
          pltpu.VMEM((2, BLK, CHUNK), jnp.int32),
          pltpu.VMEM((2, CHUNK, D), jnp.float32),
          pltpu.VMEM((8, D), jnp.float32),
          pltpu.VMEM((CHUNK,), jnp.float32),
          pltpu.VMEM((CHUNK,), jnp.float32),
          [pltpu.SemaphoreType.DMA] * 2,
          [pltpu.SemaphoreType.DMA] * 2,
          [pltpu.SemaphoreType.DMA] * 2,
      ],
  )
  def agg(x_hbm, src_hbm, dst_hbm, sums_out, deg_out,
          sums_sh, deg_sh, sib, dib, rows, zbuf, ones_v, dz_v,
          isems, jsems, gsems):
    c = lax.axis_index("c")
    s = lax.axis_index("s")
    base = s * ROWS_PER_TILE

    zero16 = jnp.zeros((16,), jnp.float32)
    for r in range(8):
      for k in range(D // 16):
        zbuf[r, pl.ds(k * 16, 16)] = zero16
    for k in range(CHUNK // 16):
      ones_v[pl.ds(k * 16, 16)] = jnp.ones((16,), jnp.float32)
      dz_v[pl.ds(k * 16, 16)] = zero16

    def zbody(i, _):
      pltpu.sync_copy(zbuf, sums_sh.at[pl.ds(base + i * 8, 8)])
      return _
    lax.fori_loop(0, ROWS_PER_TILE // 8, zbody, None)
    for i in range(ROWS_PER_TILE // CHUNK):
      pltpu.sync_copy(dz_v, deg_sh.at[pl.ds(base + i * CHUNK, CHUNK)])

    pltpu.async_copy(src_hbm.at[c, s, 0], sib.at[0], isems[0])
    pltpu.async_copy(dst_hbm.at[c, s, 0], dib.at[0], jsems[0])

    plsc.subcore_barrier()

    def scatter(p, t, b):
      pltpu.sync_copy(rows.at[b], sums_sh.at[dib.at[p, t]], add=True)
      pltpu.sync_copy(ones_v, deg_sh.at[dib.at[p, t]], add=True)

    for m in range(NBLK):
      p = m % 2
      q = p ^ 1
      pltpu.make_async_copy(src_hbm.at[c, s, m], sib.at[p], isems[p]).wait()
      pltpu.make_async_copy(dst_hbm.at[c, s, m], dib.at[p], jsems[p]).wait()
      if m + 1 < NBLK:
        pltpu.async_copy(src_hbm.at[c, s, m + 1], sib.at[q], isems[q])
        pltpu.async_copy(dst_hbm.at[c, s, m + 1], dib.at[q], jsems[q])
      gh = [None, None]
      for t in range(BLK):
        b = t % 2
        gh[b] = pltpu.async_copy(x_hbm.at[sib.at[p, t]], rows.at[b], gsems[b])
        if t >= 1:
          gh[b ^ 1].wait()
          scatter(p, t - 1, b ^ 1)
      gh[(BLK - 1) % 2].wait()
      scatter(p, BLK - 1, (BLK - 1) % 2)

    plsc.subcore_barrier()

    pltpu.sync_copy(sums_sh.at[pl.ds(base, ROWS_PER_TILE)],
                    sums_out.at[c, pl.ds(base, ROWS_PER_TILE)])
    pltpu.sync_copy(deg_sh.at[pl.ds(base, ROWS_PER_TILE)],
                    deg_out.at[c, pl.ds(base, ROWS_PER_TILE)])

  return agg(x, src_p, dst_p)


def _tc_finish(s0, s1, d0, d1, W, b):
  BN = 1024
  grid = (NPAD // BN,)

  def tc_body(s0_ref, s1_ref, d0_ref, d1_ref, w_ref, b_ref, out_ref):
    ssum = s0_ref[...] + s1_ref[...]
    deg = jnp.maximum(d0_ref[...] + d1_ref[...], 1.0)
    h = ssum / deg
    out_ref[...] = (
        jnp.dot(h, w_ref[...], preferred_element_type=jnp.float32) + b_ref[...])

  return pl.pallas_call(
      tc_body,
      grid=grid,
      in_specs=[
          pl.BlockSpec((BN, D), lambda i: (i, 0)),
          pl.BlockSpec((BN, D), lambda i: (i, 0)),
          pl.BlockSpec((BN, 1), lambda i: (i, 0)),
          pl.BlockSpec((BN, 1), lambda i: (i, 0)),
          pl.BlockSpec((D, D), lambda i: (0, 0)),
          pl.BlockSpec((1, D), lambda i: (0, 0)),
      ],
      out_specs=pl.BlockSpec((BN, D), lambda i: (i, 0)),
      out_shape=jax.ShapeDtypeStruct((NPAD, D), jnp.float32),
  )(s0, s1, d0, d1, W, b)


def kernel(x, edge_index, W_neigh, b_neigh):
  src = edge_index[0]
  dst = edge_index[1]
  pad = E_PAD - N_EDGES
  pad_src = jnp.zeros((pad,), jnp.int32)
  pad_dst = N_NODES + (jnp.arange(pad, dtype=jnp.int32) % (NPAD - N_NODES))
  src_p = jnp.concatenate([src, pad_src]).reshape(NC, NS, NBLK, BLK, CHUNK)
  dst_p = jnp.concatenate([dst, pad_dst]).reshape(NC, NS, NBLK, BLK, CHUNK)

  sums, deg = _sc_aggregate(x, src_p, dst_p)
  out = _tc_finish(
      sums[0], sums[1],
      deg[0].reshape(NPAD, 1), deg[1].reshape(NPAD, 1),
      W_neigh, b_neigh.reshape(1, D))
  return out[:N_NODES]


# speedup vs baseline: 12.5627x; 3.1376x over previous
import functools

import jax
import jax.numpy as jnp
from jax import lax
from jax.experimental import pallas as pl
from jax.experimental.pallas import tpu as pltpu
from jax.experimental.pallas import tpu_sc as plsc

N_NODES = 10000
N_EDGES = 320000
D = 128

NC = 2
NS = 16
CHUNK = 125
BLK = 16
NBLK = 5
NPAD = 10240
ROWS_PER_TILE = NPAD // NS
assert NC * NS * NBLK * BLK * CHUNK == N_EDGES


def _sc_aggregate(x, src_p, dst_p):
  mesh = plsc.VectorSubcoreMesh(core_axis_name="c", subcore_axis_name="s")

  @functools.partial(
      pl.kernel,
      out_type=[
          jax.ShapeDtypeStruct((NC, NPAD, D), jnp.float32),
          jax.ShapeDtypeStruct((NC, NPAD), jnp.float32),
      ],
      mesh=mesh,
      scratch_types=[
          pltpu.VMEM_SHARED((NPAD, D), jnp.float32),
          pltpu.VMEM_SHARED((NPAD,), jnp.float32),
          pltpu.VMEM((2, BLK, CHUNK), jnp.int32),
          pltpu.VMEM((2, BLK, CHUNK), jnp.int32),
          pltpu.VMEM((2, CHUNK, D), jnp.float32),
          pltpu.VMEM((8, D), jnp.float32),
          pltpu.VMEM((128,), jnp.float32),
          pltpu.VMEM((128,), jnp.float32),
          [pltpu.SemaphoreType.DMA] * 2,
          [pltpu.SemaphoreType.DMA] * 2,
          [pltpu.SemaphoreType.DMA] * 2,
      ],
  )
  def agg(x_hbm, src_hbm, dst_hbm, sums_out, deg_out,
          sums_sh, deg_sh, sib, dib, rows, zbuf, ones_v, dz_v,
          isems, jsems, gsems):
    c = lax.axis_index("c")
    s = lax.axis_index("s")
    base = s * ROWS_PER_TILE

    with jax.named_scope("zero_phase"):
      zero16 = jnp.zeros((16,), jnp.float32)
      for r in range(8):
        for k in range(D // 16):
          zbuf[r, pl.ds(k * 16, 16)] = zero16
      for k in range(128 // 16):
        ones_v[pl.ds(k * 16, 16)] = jnp.ones((16,), jnp.float32)
        dz_v[pl.ds(k * 16, 16)] = zero16

      def zbody(i, _):
        pltpu.sync_copy(zbuf, sums_sh.at[pl.ds(base + i * 8, 8)])
        return _
      lax.fori_loop(0, ROWS_PER_TILE // 8, zbody, None)
      for i in range(ROWS_PER_TILE // 128):
        pltpu.sync_copy(dz_v, deg_sh.at[pl.ds(base + i * 128, 128)])

      pltpu.async_copy(src_hbm.at[c, s, 0], sib.at[0], isems[0])
      pltpu.async_copy(dst_hbm.at[c, s, 0], dib.at[0], jsems[0])

      plsc.subcore_barrier()

    def scatter(p, t, b):
      pltpu.sync_copy(rows.at[b], sums_sh.at[dib.at[p, t]], add=True)
      pltpu.sync_copy(ones_v.at[pl.ds(0, CHUNK)], deg_sh.at[dib.at[p, t]], add=True)

    main_scope = jax.named_scope("main_phase")
    main_scope.__enter__()
    for m in range(NBLK):
      p = m % 2
      q = p ^ 1
      pltpu.make_async_copy(src_hbm.at[c, s, m], sib.at[p], isems[p]).wait()
      pltpu.make_async_copy(dst_hbm.at[c, s, m], dib.at[p], jsems[p]).wait()
      if m + 1 < NBLK:
        pltpu.async_copy(src_hbm.at[c, s, m + 1], sib.at[q], isems[q])
        pltpu.async_copy(dst_hbm.at[c, s, m + 1], dib.at[q], jsems[q])
      gh = [None, None]
      for t in range(BLK):
        b = t % 2
        gh[b] = pltpu.async_copy(x_hbm.at[sib.at[p, t]], rows.at[b], gsems[b])
        if t >= 1:
          gh[b ^ 1].wait()
          scatter(p, t - 1, b ^ 1)
      gh[(BLK - 1) % 2].wait()
      scatter(p, BLK - 1, (BLK - 1) % 2)

    main_scope.__exit__(None, None, None)

    with jax.named_scope("drain_phase"):
      plsc.subcore_barrier()

      pltpu.sync_copy(sums_sh.at[pl.ds(base, ROWS_PER_TILE)],
                      sums_out.at[c, pl.ds(base, ROWS_PER_TILE)])
      pltpu.sync_copy(deg_sh.at[pl.ds(base, ROWS_PER_TILE)],
                      deg_out.at[c, pl.ds(base, ROWS_PER_TILE)])

  return agg(x, src_p, dst_p)


def _tc_finish(s0, s1, d0, d1, W, b):
  BN = 1024
  grid = (NPAD // BN,)

  def tc_body(s0_ref, s1_ref, d0_ref, d1_ref, w_ref, b_ref, out_ref):
    ssum = s0_ref[...] + s1_ref[...]
    deg = jnp.maximum(d0_ref[...] + d1_ref[...], 1.0)
    h = ssum / deg
    out_ref[...] = (
        jnp.dot(h, w_ref[...], preferred_element_type=jnp.float32) + b_ref[...])

  return pl.pallas_call(
      tc_body,
      grid=grid,
      in_specs=[
          pl.BlockSpec((BN, D), lambda i: (i, 0)),
          pl.BlockSpec((BN, D), lambda i: (i, 0)),
          pl.BlockSpec((BN, 1), lambda i: (i, 0)),
          pl.BlockSpec((BN, 1), lambda i: (i, 0)),
          pl.BlockSpec((D, D), lambda i: (0, 0)),
          pl.BlockSpec((1, D), lambda i: (0, 0)),
      ],
      out_specs=pl.BlockSpec((BN, D), lambda i: (i, 0)),
      out_shape=jax.ShapeDtypeStruct((NPAD, D), jnp.float32),
  )(s0, s1, d0, d1, W, b)


def kernel(x, edge_index, W_neigh, b_neigh):
  src = edge_index[0]
  dst = edge_index[1]
  src_p = src.reshape(NC, NS, NBLK, BLK, CHUNK)
  dst_p = dst.reshape(NC, NS, NBLK, BLK, CHUNK)

  sums, deg = _sc_aggregate(x, src_p, dst_p)
  out = _tc_finish(
      sums[0], sums[1],
      deg[0].reshape(NPAD, 1), deg[1].reshape(NPAD, 1),
      W_neigh, b_neigh.reshape(1, D))
  return out[:N_NODES]


# speedup vs baseline: 15.0474x; 1.1978x over previous
import functools

import jax
import jax.numpy as jnp
from jax import lax
from jax.experimental import pallas as pl
from jax.experimental.pallas import tpu as pltpu
from jax.experimental.pallas import tpu_sc as plsc

N_NODES = 10000
N_EDGES = 320000
D = 128

NC = 2
NS = 16
CHUNK = 125
BLK = 16
NBLK = 5
NPAD = 10240
ROWS_PER_TILE = NPAD // NS
assert NC * NS * NBLK * BLK * CHUNK == N_EDGES


def _sc_aggregate(x, edges):
  mesh = plsc.VectorSubcoreMesh(core_axis_name="c", subcore_axis_name="s")

  @functools.partial(
      pl.kernel,
      out_type=[
          jax.ShapeDtypeStruct((NC, NPAD, D), jnp.float32),
          jax.ShapeDtypeStruct((NC, NPAD), jnp.float32),
      ],
      mesh=mesh,
      scratch_types=[
          pltpu.VMEM_SHARED((NPAD, D), jnp.float32),
          pltpu.VMEM_SHARED((NPAD,), jnp.float32),
          pltpu.VMEM((2, BLK, CHUNK), jnp.int32),
          pltpu.VMEM((2, BLK, CHUNK), jnp.int32),
          pltpu.VMEM((2, CHUNK, D), jnp.float32),
          pltpu.VMEM((8, D), jnp.float32),
          pltpu.VMEM((128,), jnp.float32),
          pltpu.VMEM((128,), jnp.float32),
          [pltpu.SemaphoreType.DMA] * 2,
          [pltpu.SemaphoreType.DMA] * 2,
          [pltpu.SemaphoreType.DMA] * 2,
          [pltpu.SemaphoreType.DMA] * 2,
          [pltpu.SemaphoreType.DMA] * 2,
      ],
  )
  def agg(x_hbm, e_hbm, sums_out, deg_out,
          sums_sh, deg_sh, sib, dib, rows, zbuf, ones_v, dz_v,
          isems, jsems, gsems, ssems, dsems):
    c = lax.axis_index("c")
    s = lax.axis_index("s")
    tid = c * NS + s
    base = s * ROWS_PER_TILE

    with jax.named_scope("zero_phase"):
      zero16 = jnp.zeros((16,), jnp.float32)
      for r in range(8):
        for k in range(D // 16):
          zbuf[r, pl.ds(k * 16, 16)] = zero16
      for k in range(128 // 16):
        ones_v[pl.ds(k * 16, 16)] = jnp.ones((16,), jnp.float32)
        dz_v[pl.ds(k * 16, 16)] = zero16

      def zbody(i, _):
        pltpu.sync_copy(zbuf, sums_sh.at[pl.ds(base + i * 8, 8)])
        return _
      lax.fori_loop(0, ROWS_PER_TILE // 8, zbody, None)
      for i in range(ROWS_PER_TILE // 128):
        pltpu.sync_copy(dz_v, deg_sh.at[pl.ds(base + i * 128, 128)])

      pltpu.async_copy(e_hbm.at[0, tid, 0], sib.at[0], isems[0])
      pltpu.async_copy(e_hbm.at[1, tid, 0], dib.at[0], jsems[0])

      plsc.subcore_barrier()

    main_scope = jax.named_scope("main_phase")
    main_scope.__enter__()
    gh = [None, None]
    sh = [None, None]
    dh = [None, None]
    prev = None
    for m in range(NBLK):
      p = m % 2
      q = p ^ 1
      pltpu.make_async_copy(e_hbm.at[0, tid, m], sib.at[p], isems[p]).wait()
      pltpu.make_async_copy(e_hbm.at[1, tid, m], dib.at[p], jsems[p]).wait()
      for t in range(BLK):
        b = t % 2
        if sh[b] is not None:
          sh[b].wait()
          dh[b].wait()
        if t == 2 and m + 1 < NBLK:
          pltpu.async_copy(e_hbm.at[0, tid, m + 1], sib.at[q], isems[q])
          pltpu.async_copy(e_hbm.at[1, tid, m + 1], dib.at[q], jsems[q])
        gh[b] = pltpu.async_copy(x_hbm.at[sib.at[p, t]], rows.at[b], gsems[b])
        if prev is not None:
          pp, pt, pb = prev
          gh[pb].wait()
          sh[pb] = pltpu.async_copy(
              rows.at[pb], sums_sh.at[dib.at[pp, pt]], ssems[pb], add=True)
          dh[pb] = pltpu.async_copy(
              ones_v.at[pl.ds(0, CHUNK)], deg_sh.at[dib.at[pp, pt]],
              dsems[pb], add=True)
        prev = (p, t, b)
    pp, pt, pb = prev
    gh[pb].wait()
    sh[pb] = pltpu.async_copy(
        rows.at[pb], sums_sh.at[dib.at[pp, pt]], ssems[pb], add=True)
    dh[pb] = pltpu.async_copy(
        ones_v.at[pl.ds(0, CHUNK)], deg_sh.at[dib.at[pp, pt]],
        dsems[pb], add=True)
    for b in (0, 1):
      if sh[b] is not None:
        sh[b].wait()
        dh[b].wait()

    main_scope.__exit__(None, None, None)

    with jax.named_scope("drain_phase"):
      plsc.subcore_barrier()

      pltpu.sync_copy(sums_sh.at[pl.ds(base, ROWS_PER_TILE)],
                      sums_out.at[c, pl.ds(base, ROWS_PER_TILE)])
      pltpu.sync_copy(deg_sh.at[pl.ds(base, ROWS_PER_TILE)],
                      deg_out.at[c, pl.ds(base, ROWS_PER_TILE)])

  return agg(x, edges)


def _tc_finish(sums, deg, W, b):
  BN = 2000
  grid = (N_NODES // BN,)

  def tc_body(sums_ref, deg_ref, w_ref, b_ref, out_ref):
    ssum = sums_ref[0] + sums_ref[1]
    d = jnp.maximum(deg_ref[0] + deg_ref[1], 1.0)
    h = ssum / d
    out_ref[...] = (
        jnp.dot(h, w_ref[...], preferred_element_type=jnp.float32) + b_ref[...])

  return pl.pallas_call(
      tc_body,
      grid=grid,
      in_specs=[
          pl.BlockSpec((NC, BN, D), lambda i: (0, i, 0)),
          pl.BlockSpec((NC, BN, 1), lambda i: (0, i, 0)),
          pl.BlockSpec((D, D), lambda i: (0, 0)),
          pl.BlockSpec((1, D), lambda i: (0, 0)),
      ],
      out_specs=pl.BlockSpec((BN, D), lambda i: (i, 0)),
      out_shape=jax.ShapeDtypeStruct((N_NODES, D), jnp.float32),
  )(sums, deg, W, b)


def kernel(x, edge_index, W_neigh, b_neigh):
  edges = edge_index.reshape(2, NC * NS, NBLK, BLK, CHUNK)

  sums, deg = _sc_aggregate(x, edges)
  return _tc_finish(sums, deg.reshape(NC, NPAD, 1),
                    W_neigh, b_neigh.reshape(1, D))


# speedup vs baseline: 15.2956x; 1.0165x over previous
import functools

import jax
import jax.numpy as jnp
from jax import lax
from jax.experimental import pallas as pl
from jax.experimental.pallas import tpu as pltpu
from jax.experimental.pallas import tpu_sc as plsc

N_NODES = 10000
N_EDGES = 320000
D = 128

NC = 2
NS = 16
CHUNK = 125
BLK = 16
NBLK = 5
NPAD = 10240
ROWS_PER_TILE = NPAD // NS
assert NC * NS * NBLK * BLK * CHUNK == N_EDGES


def _sc_aggregate(x, edges):
  mesh = plsc.VectorSubcoreMesh(core_axis_name="c", subcore_axis_name="s")

  @functools.partial(
      pl.kernel,
      out_type=[
          jax.ShapeDtypeStruct((NC, NPAD, D), jnp.float32),
          jax.ShapeDtypeStruct((NC, NPAD), jnp.float32),
      ],
      mesh=mesh,
      scratch_types=[
          pltpu.VMEM_SHARED((NPAD, D), jnp.float32),
          pltpu.VMEM_SHARED((NPAD,), jnp.float32),
          pltpu.VMEM((2, BLK, CHUNK), jnp.int32),
          pltpu.VMEM((2, BLK, CHUNK), jnp.int32),
          pltpu.VMEM((2, CHUNK, D), jnp.float32),
          pltpu.VMEM((16, D), jnp.float32),
          pltpu.VMEM((128,), jnp.float32),
          pltpu.VMEM((128,), jnp.float32),
          [pltpu.SemaphoreType.DMA] * 2,
          [pltpu.SemaphoreType.DMA] * 2,
          [pltpu.SemaphoreType.DMA] * 2,
          [pltpu.SemaphoreType.DMA] * 2,
          [pltpu.SemaphoreType.DMA] * 2,
      ],
  )
  def agg(x_hbm, e_hbm, sums_out, deg_out,
          sums_sh, deg_sh, sib, dib, rows, zbuf, ones_v, dz_v,
          isems, jsems, gsems, ssems, dsems):
    c = lax.axis_index("c")
    s = lax.axis_index("s")
    tid = c * NS + s
    base = s * ROWS_PER_TILE

    with jax.named_scope("zero_phase"):
      zero16 = jnp.zeros((16,), jnp.float32)
      for r in range(16):
        for k in range(D // 16):
          zbuf[r, pl.ds(k * 16, 16)] = zero16
      for k in range(128 // 16):
        ones_v[pl.ds(k * 16, 16)] = jnp.ones((16,), jnp.float32)
        dz_v[pl.ds(k * 16, 16)] = zero16

      def zbody(i, _):
        pltpu.sync_copy(zbuf, sums_sh.at[pl.ds(base + i * 16, 16)])
        return _
      lax.fori_loop(0, ROWS_PER_TILE // 16, zbody, None)
      for i in range(ROWS_PER_TILE // 128):
        pltpu.sync_copy(dz_v, deg_sh.at[pl.ds(base + i * 128, 128)])

      pltpu.async_copy(e_hbm.at[0, tid, 0], sib.at[0], isems[0])
      pltpu.async_copy(e_hbm.at[1, tid, 0], dib.at[0], jsems[0])

      plsc.subcore_barrier()

    main_scope = jax.named_scope("main_phase")
    main_scope.__enter__()
    gh = [None, None]
    sh = [None, None]
    dh = [None, None]
    prev = None
    for m in range(NBLK):
      p = m % 2
      q = p ^ 1
      pltpu.make_async_copy(e_hbm.at[0, tid, m], sib.at[p], isems[p]).wait()
      pltpu.make_async_copy(e_hbm.at[1, tid, m], dib.at[p], jsems[p]).wait()
      for t in range(BLK):
        b = t % 2
        if sh[b] is not None:
          sh[b].wait()
          dh[b].wait()
        if t == 2 and m + 1 < NBLK:
          pltpu.async_copy(e_hbm.at[0, tid, m + 1], sib.at[q], isems[q])
          pltpu.async_copy(e_hbm.at[1, tid, m + 1], dib.at[q], jsems[q])
        gh[b] = pltpu.async_copy(x_hbm.at[sib.at[p, t]], rows.at[b], gsems[b])
        if prev is not None:
          pp, pt, pb = prev
          gh[pb].wait()
          sh[pb] = pltpu.async_copy(
              rows.at[pb], sums_sh.at[dib.at[pp, pt]], ssems[pb], add=True)
          dh[pb] = pltpu.async_copy(
              ones_v.at[pl.ds(0, CHUNK)], deg_sh.at[dib.at[pp, pt]],
              dsems[pb], add=True)
        prev = (p, t, b)
    pp, pt, pb = prev
    gh[pb].wait()
    sh[pb] = pltpu.async_copy(
        rows.at[pb], sums_sh.at[dib.at[pp, pt]], ssems[pb], add=True)
    dh[pb] = pltpu.async_copy(
        ones_v.at[pl.ds(0, CHUNK)], deg_sh.at[dib.at[pp, pt]],
        dsems[pb], add=True)
    for b in (0, 1):
      if sh[b] is not None:
        sh[b].wait()
        dh[b].wait()

    main_scope.__exit__(None, None, None)

    with jax.named_scope("drain_phase"):
      plsc.subcore_barrier()

      pltpu.sync_copy(sums_sh.at[pl.ds(base, ROWS_PER_TILE)],
                      sums_out.at[c, pl.ds(base, ROWS_PER_TILE)])
      pltpu.sync_copy(deg_sh.at[pl.ds(base, ROWS_PER_TILE)],
                      deg_out.at[c, pl.ds(base, ROWS_PER_TILE)])

  return agg(x, edges)


def _tc_finish(sums, deg, W, b):
  BN = 5000
  grid = (N_NODES // BN,)

  def tc_body(sums_ref, deg_ref, w_ref, b_ref, out_ref):
    ssum = sums_ref[0] + sums_ref[1]
    d = jnp.maximum(deg_ref[0] + deg_ref[1], 1.0)
    h = ssum / d
    out_ref[...] = (
        jnp.dot(h, w_ref[...], preferred_element_type=jnp.float32) + b_ref[...])

  return pl.pallas_call(
      tc_body,
      grid=grid,
      in_specs=[
          pl.BlockSpec((NC, BN, D), lambda i: (0, i, 0)),
          pl.BlockSpec((NC, BN, 1), lambda i: (0, i, 0)),
          pl.BlockSpec((D, D), lambda i: (0, 0)),
          pl.BlockSpec((1, D), lambda i: (0, 0)),
      ],
      out_specs=pl.BlockSpec((BN, D), lambda i: (i, 0)),
      out_shape=jax.ShapeDtypeStruct((N_NODES, D), jnp.float32),
  )(sums, deg, W, b)


def kernel(x, edge_index, W_neigh, b_neigh):
  edges = edge_index.reshape(2, NC * NS, NBLK, BLK, CHUNK)

  sums, deg = _sc_aggregate(x, edges)
  return _tc_finish(sums, deg.reshape(NC, NPAD, 1),
                    W_neigh, b_neigh.reshape(1, D))
